# baseline (speedup 1.0000x reference)
import functools

import jax
import jax.numpy as jnp
from jax import lax
from jax.experimental import pallas as pl
from jax.experimental.pallas import tpu as pltpu
from jax.experimental.pallas import tpu_sc as plsc


@functools.cache
def _make_sc_embed(BS: int, V: int, D: int, S: int):
    info = plsc.get_sparse_core_info()
    NC, NS, L = info.num_cores, info.num_subcores, info.num_lanes
    NW = NC * NS
    assert BS % NW == 0
    b_per_w = BS // NW
    CHUNK = 32
    assert b_per_w % CHUNK == 0 and S % b_per_w == 0
    n_chunks = b_per_w // CHUNK
    n_vecs = D // L
    mesh = plsc.VectorSubcoreMesh(core_axis_name="c", subcore_axis_name="s")

    @functools.partial(
        pl.kernel,
        mesh=mesh,
        out_type=jax.ShapeDtypeStruct((BS, D), jnp.float32),
        scratch_types=[
            pltpu.VMEM((b_per_w,), jnp.int32),
            pltpu.VMEM((CHUNK, D), jnp.float32),
            pltpu.VMEM((CHUNK, D), jnp.float32),
            pltpu.SemaphoreType.DMA,
        ],
    )
    def emb(idx_hbm, table_hbm, pos_hbm, out_hbm, idx_v, rows_v, pos_v, sem):
        wid = lax.axis_index("s") * NC + lax.axis_index("c")
        base = wid * b_per_w
        pos_base = lax.rem(base, S)
        pltpu.sync_copy(idx_hbm.at[pl.ds(base, b_per_w)], idx_v)

        def chunk_body(ci, carry):
            off = pl.multiple_of(ci * CHUNK, CHUNK)
            pltpu.async_copy(
                table_hbm.at[idx_v.at[pl.ds(off, CHUNK)]], rows_v, sem
            ).wait()
            pltpu.sync_copy(pos_hbm.at[pl.ds(pos_base + off, CHUNK)], pos_v)

            def row_body(r, c2):
                def vec_body(j, c3):
                    sl = pl.ds(j * L, L)
                    plsc.addupdate(rows_v.at[r, sl], pos_v[r, sl])
                    return c3

                return lax.fori_loop(0, n_vecs, vec_body, c2)

            lax.fori_loop(0, CHUNK, row_body, 0)
            pltpu.sync_copy(rows_v, out_hbm.at[pl.ds(base + off, CHUNK)])
            return carry

        lax.fori_loop(0, n_chunks, chunk_body, 0)

    return emb


def kernel(input_ids, word_embeddings, position_embeddings):
    B, S = input_ids.shape
    V, D = word_embeddings.shape
    ids_flat = input_ids.reshape(-1).astype(jnp.int32)
    emb = _make_sc_embed(B * S, V, D, S)
    out = emb(ids_flat, word_embeddings, position_embeddings)
    return out.reshape(B, S, D)


# speedup vs baseline: 2.9180x; 2.9180x over previous
import functools

import jax
import jax.numpy as jnp
from jax import lax
from jax.experimental import pallas as pl
from jax.experimental.pallas import tpu as pltpu
from jax.experimental.pallas import tpu_sc as plsc


@functools.cache
def _make_sc_embed(B: int, S: int, V: int, D: int):
    info = plsc.get_sparse_core_info()
    NC, NS, L = info.num_cores, info.num_subcores, info.num_lanes
    NW = NC * NS
    assert S % NW == 0
    s_per_w = S // NW
    CHUNK = 32
    assert s_per_w % CHUNK == 0
    n_sc = s_per_w // CHUNK
    n_steps = n_sc * B
    vecs = CHUNK * D // L
    UNROLL = 4
    assert vecs % UNROLL == 0
    mesh = plsc.VectorSubcoreMesh(core_axis_name="c", subcore_axis_name="s")

    @functools.partial(
        pl.kernel,
        mesh=mesh,
        out_type=jax.ShapeDtypeStruct((B * S, D), jnp.float32),
        scratch_types=[
            pltpu.VMEM((B * s_per_w,), jnp.int32),
            pltpu.VMEM((CHUNK, D), jnp.float32),
            pltpu.VMEM((CHUNK, D), jnp.float32),
            pltpu.VMEM((CHUNK, D), jnp.float32),
            pltpu.SemaphoreType.DMA,
            pltpu.SemaphoreType.DMA,
            pltpu.SemaphoreType.DMA,
            pltpu.SemaphoreType.DMA,
        ],
    )
    def emb(idx_hbm, table_hbm, pos_hbm, out_hbm,
            idx_v, w0, w1, pos_v, g0, g1, o0, o1):
        wid = lax.axis_index("s") * NC + lax.axis_index("c")
        s_base = wid * s_per_w
        wbuf = (w0, w1)
        gsem = (g0, g1)
        osem = (o0, o1)

        for b in range(B):
            pltpu.sync_copy(
                idx_hbm.at[pl.ds(b * S + s_base, s_per_w)],
                idx_v.at[pl.ds(b * s_per_w, s_per_w)],
            )

        def gather(k, buf):
            sc, b = divmod(k, B)
            off = b * s_per_w + sc * CHUNK
            return pltpu.async_copy(
                table_hbm.at[idx_v.at[pl.ds(off, CHUNK)]],
                wbuf[buf], gsem[buf],
            )

        def add_pos(buf):
            cur = wbuf[buf]

            def body(i, carry):
                for u in range(UNROLL):
                    t = i * UNROLL + u
                    r = lax.shift_right_logical(t, 6)
                    j = lax.shift_left(lax.bitwise_and(t, 63), 4)
                    sl = pl.ds(pl.multiple_of(j, L), L)
                    plsc.addupdate(cur.at[r, sl], pos_v[r, sl])
                return carry

            lax.fori_loop(0, vecs // UNROLL, body, 0)

        pending_g = gather(0, 0)
        pending_o = [None, None]
        for k in range(n_steps):
            sc, b = divmod(k, B)
            cur = k % 2
            nxt = (k + 1) % 2
            if b == 0:
                pltpu.sync_copy(
                    pos_hbm.at[pl.ds(s_base + sc * CHUNK, CHUNK)], pos_v
                )
            if k + 1 < n_steps:
                if pending_o[nxt] is not None:
                    pending_o[nxt].wait()
                    pending_o[nxt] = None
                next_g = gather(k + 1, nxt)
            pending_g.wait()
            add_pos(cur)
            pending_o[cur] = pltpu.async_copy(
                wbuf[cur],
                out_hbm.at[pl.ds(b * S + s_base + sc * CHUNK, CHUNK)],
                osem[cur],
            )
            if k + 1 < n_steps:
                pending_g = next_g
        for d in pending_o:
            if d is not None:
                d.wait()

    return emb


def kernel(input_ids, word_embeddings, position_embeddings):
    B, S = input_ids.shape
    V, D = word_embeddings.shape
    ids_flat = input_ids.reshape(-1).astype(jnp.int32)
    emb = _make_sc_embed(B, S, V, D)
    out = emb(ids_flat, word_embeddings, position_embeddings)
    return out.reshape(B, S, D)
